# baseline (speedup 1.0000x reference)
import functools

import jax
import jax.numpy as jnp
from jax import lax
from jax.experimental import pallas as pl
from jax.experimental.pallas import tpu as pltpu
from jax.experimental.pallas import tpu_sc as plsc

_N = 10000
_D = 128
_E = 320000
_NC = 2
_NS = 16
_NW = _NC * _NS
_K = 128
_CH = -(-_E // (_NW * _K))
_EP = _CH * _K
_P = _EP * _NW
_NA = 10240
_ZR = _NA // _NS
_LAG = 4
_EPT = _E // _NW
_CHF = _EPT // _K
_TL = _EPT - _CHF * _K
_NDUM = _NA - _N


def _sc_mesh():
    return plsc.VectorSubcoreMesh(
        core_axis_name="c", subcore_axis_name="s",
        num_cores=_NC, num_subcores=_NS)


@functools.cache
def _sc_degree_kernel():
    return pl.kernel(
        _sc_degree_body,
        out_type=jax.ShapeDtypeStruct((_NC, _NA), jnp.float32),
        mesh=_sc_mesh(),
        scratch_types=[
            pltpu.VMEM_SHARED((_NA,), jnp.float32),
            pltpu.VMEM((_CHF * _K,), jnp.int32),
            pltpu.VMEM((_K,), jnp.int32),
            pltpu.VMEM((_K,), jnp.float32),
            pltpu.SemaphoreType.DMA,
        ],
    )


def _sc_degree_body(ei_hbm, z_hbm, out_hbm, deg_sh, sidx, sidx_t, ones_v, sem):
    c = lax.axis_index("c")
    s = lax.axis_index("s")
    w = c * _NS + s
    for j in range(_K // 16):
        ones_v[pl.ds(j * 16, 16)] = jnp.ones((16,), jnp.float32)
    lanes = lax.iota(jnp.int32, 16)
    for j in range(_K // 16):
        sidx_t[pl.ds(j * 16, 16)] = _N + (j * 16 + lanes + s * 13) % _NDUM
    pltpu.sync_copy(z_hbm, deg_sh.at[pl.ds(s * _ZR, _ZR)])
    pltpu.sync_copy(ei_hbm.at[pl.ds(w * _EPT, _CHF * _K)], sidx)
    pltpu.sync_copy(ei_hbm.at[pl.ds(w * _EPT + _CHF * _K, _TL)],
                    sidx_t.at[pl.ds(0, _TL)])
    plsc.subcore_barrier()

    @pl.loop(0, _CHF)
    def _fire(i):
        @pl.when(i >= _LAG)
        def _lagged_drain():
            pltpu.make_async_copy(ones_v, deg_sh.at[sidx.at[pl.ds(0, _K)]], sem).wait()
        pltpu.async_copy(ones_v, deg_sh.at[sidx.at[pl.ds(i * _K, _K)]], sem, add=True)

    pltpu.async_copy(ones_v, deg_sh.at[sidx_t], sem, add=True)

    @pl.loop(0, min(_LAG, _CHF) + 1)
    def _drain(i):
        pltpu.make_async_copy(ones_v, deg_sh.at[sidx.at[pl.ds(0, _K)]], sem).wait()

    plsc.subcore_barrier()
    pltpu.sync_copy(deg_sh.at[pl.ds(s * _ZR, _ZR)], out_hbm.at[c, pl.ds(s * _ZR, _ZR)])


_DN = 4


@functools.cache
def _sc_aggregate_kernel():
    return pl.kernel(
        _sc_aggregate_body,
        out_type=jax.ShapeDtypeStruct((_NC, _NA, _D), jnp.float32),
        mesh=_sc_mesh(),
        scratch_types=[
            pltpu.VMEM_SHARED((_NA, _D), jnp.float32),
            pltpu.VMEM((_CH, _K), jnp.int32),
            pltpu.VMEM((_DN, _K), jnp.int32),
            pltpu.VMEM((2, _K, _D), jnp.float32),
            pltpu.SemaphoreType.DMA,
            pltpu.SemaphoreType.DMA,
        ] + [pltpu.SemaphoreType.DMA] * _DN,
    )


def _sc_aggregate_body(y_hbm, sp_hbm, dp_hbm, z_hbm, out_hbm,
                       acc_sh, sidx, didx, rows, gsem0, gsem1, *dsems):
    c = lax.axis_index("c")
    s = lax.axis_index("s")
    w = c * _NS + s
    pltpu.sync_copy(sp_hbm.at[w], sidx)
    pltpu.sync_copy(z_hbm, acc_sh.at[pl.ds(s * _ZR, _ZR)])

    gsems = (gsem0, gsem1)

    def _start_gather(i, b):
        pltpu.async_copy(y_hbm.at[sidx.at[i]], rows.at[b], gsems[b])

    def _wait_gather(b):
        pltpu.make_async_copy(y_hbm.at[sidx.at[0]], rows.at[b], gsems[b]).wait()

    def _fire_didx(i, sl):
        pltpu.async_copy(dp_hbm.at[w, i], didx.at[sl], dsems[sl])

    def _wait_didx(sl):
        pltpu.make_async_copy(dp_hbm.at[w, 0], didx.at[sl], dsems[sl]).wait()

    for j in range(_DN):
        _fire_didx(j, j)
    plsc.subcore_barrier()

    _start_gather(0, 0)

    @pl.loop(0, _CH, step=_DN)
    def _chunk(i):
        for b in range(_DN):
            j = i + b
            gb = b % 2
            @pl.when(j < _CH)
            def _step():
                @pl.when(j + 1 < _CH)
                def _prefetch():
                    _start_gather(j + 1, 1 - gb)
                _wait_didx(b)
                _wait_gather(gb)
                pltpu.sync_copy(rows.at[gb], acc_sh.at[didx.at[b]], add=True)
                @pl.when(j + _DN < _CH)
                def _refill():
                    _fire_didx(j + _DN, b)

    plsc.subcore_barrier()
    pltpu.sync_copy(acc_sh.at[pl.ds(s * _ZR, _ZR)], out_hbm.at[c, pl.ds(s * _ZR, _ZR)])


def _d_broadcast(d_ref):
    dsum = d_ref[0:1, :] + d_ref[1:2, :] + 1.0
    db = lax.dot_general(dsum, jnp.ones((1, _D), jnp.float32),
                         (((0,), (0,)), ((), ())),
                         preferred_element_type=jnp.float32)
    return db[:_N, :]


def _tc_prescale_body(x_ref, d_ref, y_ref):
    db = _d_broadcast(d_ref)
    y_ref[...] = x_ref[...] * lax.rsqrt(db)


def _tc_prescale(x, deg2):
    return pl.pallas_call(
        _tc_prescale_body,
        out_shape=jax.ShapeDtypeStruct((_N, _D), jnp.float32),
    )(x, deg2)


def _tc_combine_body(x_ref, d_ref, a_ref, o_ref):
    db = _d_broadcast(d_ref)
    a = a_ref[0, :_N, :] + a_ref[1, :_N, :]
    xv = x_ref[...]
    o_ref[:, :_D] = xv
    o_ref[:, _D:] = a * lax.rsqrt(db) + xv / db


def _tc_combine(x, deg2, acc2):
    return pl.pallas_call(
        _tc_combine_body,
        out_shape=jax.ShapeDtypeStruct((_N, 2 * _D), jnp.float32),
    )(x, deg2, acc2)


def kernel(x, edge_index, features_idx):
    em = edge_index.reshape(2 * _E)
    zeros1 = jnp.zeros((_ZR,), jnp.float32)
    zeros2 = jnp.zeros((_ZR, _D), jnp.float32)

    npad = _EP - _EPT
    pad_s = jnp.broadcast_to(jnp.arange(_TL, _TL + npad, dtype=jnp.int32),
                             (_NW, npad))
    pad_d = _N + (jnp.arange(npad, dtype=jnp.int32)[None, :]
                  + 13 * jnp.arange(_NW, dtype=jnp.int32)[:, None]) % _NDUM
    srcp = jnp.concatenate(
        [edge_index[0].reshape(_NW, _EPT), pad_s], axis=1).reshape(_NW, _CH, _K)
    dstp = jnp.concatenate(
        [edge_index[1].reshape(_NW, _EPT), pad_d], axis=1).reshape(_NW, _CH, _K)

    deg2 = _sc_degree_kernel()(em, zeros1)
    y = _tc_prescale(x, deg2)
    acc2 = _sc_aggregate_kernel()(y, srcp, dstp, zeros2)
    return _tc_combine(x, deg2, acc2)


# speedup vs baseline: 1.0035x; 1.0035x over previous
import functools

import jax
import jax.numpy as jnp
from jax import lax
from jax.experimental import pallas as pl
from jax.experimental.pallas import tpu as pltpu
from jax.experimental.pallas import tpu_sc as plsc

_N = 10000
_D = 128
_E = 320000
_NC = 2
_NS = 16
_NW = _NC * _NS
_K = 128
_CH = -(-_E // (_NW * _K))
_EP = _CH * _K
_P = _EP * _NW
_NA = 10240
_ZR = _NA // _NS
_LAG = 4
_EPT = _E // _NW
_CHF = _EPT // _K
_TL = _EPT - _CHF * _K
_NDUM = _NA - _N


def _sc_mesh():
    return plsc.VectorSubcoreMesh(
        core_axis_name="c", subcore_axis_name="s",
        num_cores=_NC, num_subcores=_NS)


@functools.cache
def _sc_degree_kernel():
    return pl.kernel(
        _sc_degree_body,
        out_type=jax.ShapeDtypeStruct((_NC, _NA), jnp.float32),
        mesh=_sc_mesh(),
        scratch_types=[
            pltpu.VMEM_SHARED((_NA,), jnp.float32),
            pltpu.VMEM((_CHF * _K,), jnp.int32),
            pltpu.VMEM((_K,), jnp.int32),
            pltpu.VMEM((_K,), jnp.float32),
            pltpu.SemaphoreType.DMA,
        ],
    )


def _sc_degree_body(ei_hbm, z_hbm, out_hbm, deg_sh, sidx, sidx_t, ones_v, sem):
    c = lax.axis_index("c")
    s = lax.axis_index("s")
    w = c * _NS + s
    for j in range(_K // 16):
        ones_v[pl.ds(j * 16, 16)] = jnp.ones((16,), jnp.float32)
    lanes = lax.iota(jnp.int32, 16)
    for j in range(_K // 16):
        sidx_t[pl.ds(j * 16, 16)] = _N + (j * 16 + lanes + s * 13) % _NDUM
    pltpu.sync_copy(z_hbm, deg_sh.at[pl.ds(s * _ZR, _ZR)])
    pltpu.sync_copy(ei_hbm.at[pl.ds(w * _EPT, _CHF * _K)], sidx)
    pltpu.sync_copy(ei_hbm.at[pl.ds(w * _EPT + _CHF * _K, _TL)],
                    sidx_t.at[pl.ds(0, _TL)])
    plsc.subcore_barrier()

    @pl.loop(0, _CHF)
    def _fire(i):
        @pl.when(i >= _LAG)
        def _lagged_drain():
            pltpu.make_async_copy(ones_v, deg_sh.at[sidx.at[pl.ds(0, _K)]], sem).wait()
        pltpu.async_copy(ones_v, deg_sh.at[sidx.at[pl.ds(i * _K, _K)]], sem, add=True)

    pltpu.async_copy(ones_v, deg_sh.at[sidx_t], sem, add=True)

    @pl.loop(0, min(_LAG, _CHF) + 1)
    def _drain(i):
        pltpu.make_async_copy(ones_v, deg_sh.at[sidx.at[pl.ds(0, _K)]], sem).wait()

    plsc.subcore_barrier()
    pltpu.sync_copy(deg_sh.at[pl.ds(s * _ZR, _ZR)], out_hbm.at[c, pl.ds(s * _ZR, _ZR)])


_DN = 4


@functools.cache
def _sc_aggregate_kernel():
    return pl.kernel(
        _sc_aggregate_body,
        out_type=jax.ShapeDtypeStruct((_NC, _NA, _D), jnp.float32),
        mesh=_sc_mesh(),
        scratch_types=[
            pltpu.VMEM_SHARED((_NA, _D), jnp.float32),
            pltpu.VMEM((_CH, _K), jnp.int32),
            pltpu.VMEM((_DN, _K), jnp.int32),
            pltpu.VMEM((2, _K, _D), jnp.float32),
            pltpu.SemaphoreType.DMA,
            pltpu.SemaphoreType.DMA,
            pltpu.SemaphoreType.DMA,
            pltpu.SemaphoreType.DMA,
        ] + [pltpu.SemaphoreType.DMA] * _DN,
    )


def _sc_aggregate_body(y_hbm, sp_hbm, dp_hbm, z_hbm, out_hbm,
                       acc_sh, sidx, didx, rows, gsem0, gsem1,
                       ssem0, ssem1, *dsems):
    c = lax.axis_index("c")
    s = lax.axis_index("s")
    w = c * _NS + s
    pltpu.sync_copy(sp_hbm.at[w], sidx)
    pltpu.sync_copy(z_hbm, acc_sh.at[pl.ds(s * _ZR, _ZR)])

    gsems = (gsem0, gsem1)
    ssems = (ssem0, ssem1)

    def _start_gather(i, b):
        pltpu.async_copy(y_hbm.at[sidx.at[i]], rows.at[b], gsems[b])

    def _wait_gather(b):
        pltpu.make_async_copy(y_hbm.at[sidx.at[0]], rows.at[b], gsems[b]).wait()

    def _fire_scatter(sl, b):
        pltpu.async_copy(rows.at[b], acc_sh.at[didx.at[sl]], ssems[b], add=True)

    def _wait_scatter(b):
        pltpu.make_async_copy(rows.at[b], acc_sh.at[didx.at[0]],
                              ssems[b]).wait()

    def _fire_didx(i, sl):
        pltpu.async_copy(dp_hbm.at[w, i], didx.at[sl], dsems[sl])

    def _wait_didx(sl):
        pltpu.make_async_copy(dp_hbm.at[w, 0], didx.at[sl], dsems[sl]).wait()

    for j in range(2):
        _fire_didx(j, j)
    plsc.subcore_barrier()

    _start_gather(0, 0)

    @pl.loop(0, _CH, step=_DN)
    def _chunk(i):
        for b in range(_DN):
            j = i + b
            gb = b % 2
            @pl.when(j < _CH)
            def _step():
                @pl.when(j + 1 < _CH)
                def _prefetch():
                    @pl.when(j >= 1)
                    def _drain_prev():
                        _wait_scatter(1 - gb)
                    _start_gather(j + 1, 1 - gb)
                _wait_didx(b)
                _wait_gather(gb)
                _fire_scatter(b, gb)
                @pl.when(j + 2 < _CH)
                def _refill():
                    _fire_didx(j + 2, (b + 2) % _DN)

    _wait_scatter(0)
    _wait_scatter(1)
    plsc.subcore_barrier()
    pltpu.sync_copy(acc_sh.at[pl.ds(s * _ZR, _ZR)], out_hbm.at[c, pl.ds(s * _ZR, _ZR)])


def _d_broadcast(d_ref):
    dsum = d_ref[0:1, :] + d_ref[1:2, :] + 1.0
    db = lax.dot_general(dsum, jnp.ones((1, _D), jnp.float32),
                         (((0,), (0,)), ((), ())),
                         preferred_element_type=jnp.float32)
    return db[:_N, :]


def _tc_prescale_body(x_ref, d_ref, y_ref):
    db = _d_broadcast(d_ref)
    y_ref[...] = x_ref[...] * lax.rsqrt(db)


def _tc_prescale(x, deg2):
    return pl.pallas_call(
        _tc_prescale_body,
        out_shape=jax.ShapeDtypeStruct((_N, _D), jnp.float32),
    )(x, deg2)


def _tc_combine_body(x_ref, d_ref, a_ref, o_ref):
    db = _d_broadcast(d_ref)
    a = a_ref[0, :_N, :] + a_ref[1, :_N, :]
    xv = x_ref[...]
    o_ref[:, :_D] = xv
    o_ref[:, _D:] = a * lax.rsqrt(db) + xv / db


def _tc_combine(x, deg2, acc2):
    return pl.pallas_call(
        _tc_combine_body,
        out_shape=jax.ShapeDtypeStruct((_N, 2 * _D), jnp.float32),
    )(x, deg2, acc2)


def kernel(x, edge_index, features_idx):
    em = edge_index.reshape(2 * _E)
    zeros1 = jnp.zeros((_ZR,), jnp.float32)
    zeros2 = jnp.zeros((_ZR, _D), jnp.float32)

    npad = _EP - _EPT
    pad_s = jnp.broadcast_to(jnp.arange(_TL, _TL + npad, dtype=jnp.int32),
                             (_NW, npad))
    pad_d = _N + (jnp.arange(npad, dtype=jnp.int32)[None, :]
                  + 13 * jnp.arange(_NW, dtype=jnp.int32)[:, None]) % _NDUM
    srcp = jnp.concatenate(
        [edge_index[0].reshape(_NW, _EPT), pad_s], axis=1).reshape(_NW, _CH, _K)
    dstp = jnp.concatenate(
        [edge_index[1].reshape(_NW, _EPT), pad_d], axis=1).reshape(_NW, _CH, _K)

    deg2 = _sc_degree_kernel()(em, zeros1)
    y = _tc_prescale(x, deg2)
    acc2 = _sc_aggregate_kernel()(y, srcp, dstp, zeros2)
    return _tc_combine(x, deg2, acc2)
